# baseline (speedup 1.0000x reference)
import functools

import jax
import jax.numpy as jnp
from jax import lax
from jax.experimental import pallas as pl
from jax.experimental.pallas import tpu as pltpu
from jax.experimental.pallas import tpu_sc as plsc

VOCAB = 100000
CONTEXT = 20
EMBED = 32
HIDDEN = 128
BATCH = 1024

NIDX = BATCH * CONTEXT


@functools.cache
def _gather_call(n_idx, embed):
    info = plsc.get_sparse_core_info()
    nc, ns = info.num_cores, info.num_subcores
    nw = nc * ns
    assert n_idx % nw == 0
    b_per_w = n_idx // nw
    mesh = plsc.VectorSubcoreMesh(core_axis_name="c", subcore_axis_name="s")

    @functools.partial(
        pl.kernel,
        mesh=mesh,
        out_type=jax.ShapeDtypeStruct((n_idx, embed), jnp.float32),
        scratch_types=[
            pltpu.VMEM((b_per_w,), jnp.int32),
            pltpu.VMEM((b_per_w, embed), jnp.float32),
            pltpu.SemaphoreType.DMA,
        ],
        compiler_params=pltpu.CompilerParams(use_tc_tiling_on_sc=False),
    )
    def gather_k(idx_hbm, table_hbm, out_hbm, idx_v, rows_v, sem):
        wid = lax.axis_index("s") * nc + lax.axis_index("c")
        base = wid * b_per_w
        pltpu.sync_copy(idx_hbm.at[pl.ds(base, b_per_w)], idx_v)
        pltpu.async_copy(table_hbm.at[idx_v], rows_v, sem).wait()
        pltpu.sync_copy(rows_v, out_hbm.at[pl.ds(base, b_per_w)])

    return gather_k


def _mlp_body(emb_ref, w1_ref, b1_ref, w2_ref, b2_ref, out_ref, hid_ref):
    @pl.when(pl.program_id(0) == 0)
    def _():
        h = jnp.dot(emb_ref[...], w1_ref[...],
                    preferred_element_type=jnp.float32)
        hid_ref[...] = jnp.maximum(h + b1_ref[...], 0.0)

    out_ref[...] = jnp.dot(hid_ref[...], w2_ref[...],
                           preferred_element_type=jnp.float32) + b2_ref[...]


def kernel(x, table, W1, b1, W2, b2):
    idx = x.reshape(-1).astype(jnp.int32)
    embedded = _gather_call(NIDX, EMBED)(idx, table)
    embedded = embedded.reshape(BATCH, CONTEXT * EMBED)

    vt = 2048
    nv = pl.cdiv(VOCAB, vt)
    logits = pl.pallas_call(
        _mlp_body,
        grid=(nv,),
        in_specs=[
            pl.BlockSpec((BATCH, CONTEXT * EMBED), lambda j: (0, 0)),
            pl.BlockSpec((CONTEXT * EMBED, HIDDEN), lambda j: (0, 0)),
            pl.BlockSpec((1, HIDDEN), lambda j: (0, 0)),
            pl.BlockSpec((HIDDEN, vt), lambda j: (0, j)),
            pl.BlockSpec((1, vt), lambda j: (0, j)),
        ],
        out_specs=pl.BlockSpec((BATCH, vt), lambda j: (0, j)),
        out_shape=jax.ShapeDtypeStruct((BATCH, VOCAB), jnp.float32),
        scratch_shapes=[pltpu.VMEM((BATCH, HIDDEN), jnp.float32)],
    )(embedded, W1, b1.reshape(1, HIDDEN), W2, b2.reshape(1, VOCAB))
    return logits
